# baseline (speedup 1.0000x reference)
import functools

import jax
import jax.numpy as jnp
from jax import lax
from jax.experimental import pallas as pl
from jax.experimental.pallas import tpu as pltpu
from jax.experimental.pallas import tpu_sc as plsc

D = 32
V = 1000000
NI = 4096
NJ = 200
B = NI * NJ
NW = 32
PER_W = B // NW
SUP = 512
N_SUP = PER_W // SUP
BLKS = SUP // 128


def _gather(ids5, table):
    mesh = plsc.VectorSubcoreMesh(core_axis_name="c", subcore_axis_name="s")

    @functools.partial(
        pl.kernel,
        mesh=mesh,
        out_type=jax.ShapeDtypeStruct((B * D,), jnp.float32),
        scratch_types=[
            pltpu.VMEM((PER_W,), jnp.int32),
            pltpu.VMEM((SUP, D), jnp.float32),
            pltpu.VMEM((SUP, D), jnp.float32),
            pltpu.VMEM((4096,), jnp.float32),
            pltpu.VMEM((4096,), jnp.float32),
            pltpu.SemaphoreType.DMA,
            pltpu.SemaphoreType.DMA,
            pltpu.SemaphoreType.DMA,
            pltpu.SemaphoreType.DMA,
        ],
        compiler_params=pltpu.CompilerParams(
            use_tc_tiling_on_sc=False, needs_layout_passes=False
        ),
    )
    def k(ids_hbm, table_hbm, out_hbm, idx_v, rb0, rb1, vt0, vt1,
          gs0, gs1, os0, os1):
        w = lax.axis_index("s") * 2 + lax.axis_index("c")
        base = w * PER_W
        rbs = (rb0, rb1)
        gss = (gs0, gs1)
        vts = (vt0, vt1)
        oss = (os0, os1)
        iota = lax.iota(jnp.int32, 16)
        iota128a = iota * 128
        iota128b = iota * 128 + 2048

        pltpu.sync_copy(ids_hbm.at[pl.ds(base, PER_W)], idx_v)

        def start_gather(s, b):
            pltpu.async_copy(
                table_hbm.at[idx_v.at[pl.ds(s * SUP, SUP)]], rbs[b], gss[b]
            )

        def wait_gather(b):
            pltpu.make_async_copy(
                table_hbm.at[idx_v.at[pl.ds(0, SUP)]], rbs[b], gss[b]
            ).wait()

        def wait_out(vb):
            pltpu.make_async_copy(
                vts[vb], out_hbm.at[pl.ds(0, 4096)], oss[vb]
            ).wait()

        idx8 = tuple(iota128a + t for t in range(8))

        def transpose_block(rb, blk, vtb):
            for r in range(128):
                row = blk * 128 + r
                ra, rt = (r >> 3) << 3, r & 7
                lo = rb[row, pl.ds(0, 16)]
                hi = rb[row, pl.ds(16, 16)]
                plsc.store_scatter(vtb.at[pl.ds(ra, 1928)], [idx8[rt]], lo)
                plsc.store_scatter(vtb.at[pl.ds(ra + 2048, 1928)],
                                   [idx8[rt]], hi)

        start_gather(0, 0)

        def outer(s2, carry):
            for b in range(2):
                s = s2 * 2 + b

                @pl.when(s + 1 < N_SUP)
                def _():
                    start_gather(s + 1, 1 - b)

                wait_gather(b)
                for blk in range(BLKS):
                    vb = blk & 1
                    if blk < 2:
                        @pl.when(s > 0)
                        def _():
                            wait_out(vb)
                    else:
                        wait_out(vb)
                    transpose_block(rbs[b], blk, vts[vb])
                    g = w * (PER_W // 128) + s * BLKS + blk
                    j = g >> 5
                    ih = g & 31
                    for dh in range(4):
                        pltpu.async_copy(
                            vts[vb].at[pl.ds(dh * 1024, 1024)],
                            out_hbm.at[pl.ds(((j * 4 + dh) * 32 + ih) * 1024,
                                             1024)],
                            oss[vb],
                        )
            return carry

        lax.fori_loop(0, N_SUP // 2, outer, 0)
        wait_out(0)
        wait_out(1)

    return k(ids5, table)


def kernel(token_ids, weight):
    ids5 = jnp.transpose(token_ids).reshape(B)
    flat = _gather(ids5, weight)
    out5 = flat.reshape(NJ, 4, NI // 128, 8, 128)
    return out5.transpose(2, 4, 0, 1, 3).reshape(NI, NJ, D)


# speedup vs baseline: 1.2603x; 1.2603x over previous
import functools

import jax
import jax.numpy as jnp
from jax import lax
from jax.experimental import pallas as pl
from jax.experimental.pallas import tpu as pltpu
from jax.experimental.pallas import tpu_sc as plsc

D = 32
V = 1000000
NI = 4096
NJ = 200
B = NI * NJ
NW = 32
PER_W = B // NW
SUP = 512
N_SUP = PER_W // SUP
BLKS = SUP // 128


def _gather(ids5, table):
    mesh = plsc.VectorSubcoreMesh(core_axis_name="c", subcore_axis_name="s")

    @functools.partial(
        pl.kernel,
        mesh=mesh,
        out_type=jax.ShapeDtypeStruct((NJ, 4, NI // 128, 8, 128), jnp.float32),
        scratch_types=[
            pltpu.VMEM((PER_W,), jnp.int32),
            pltpu.VMEM((SUP, D), jnp.float32),
            pltpu.VMEM((SUP, D), jnp.float32),
            pltpu.VMEM((D, 129), jnp.float32),
            pltpu.VMEM((D, 129), jnp.float32),
            pltpu.SemaphoreType.DMA,
            pltpu.SemaphoreType.DMA,
            pltpu.SemaphoreType.DMA,
            pltpu.SemaphoreType.DMA,
        ],
        compiler_params=pltpu.CompilerParams(
            use_tc_tiling_on_sc=False, needs_layout_passes=False
        ),
    )
    def k(ids_hbm, table_hbm, out_hbm, idx_v, rb0, rb1, vt0, vt1,
          gs0, gs1, os0, os1):
        w = lax.axis_index("s") * 2 + lax.axis_index("c")
        base = w * PER_W
        rbs = (rb0, rb1)
        gss = (gs0, gs1)
        vts = (vt0, vt1)
        oss = (os0, os1)
        iota = lax.iota(jnp.int32, 16)
        iota16p = iota + 16
        zero16 = iota - iota

        pltpu.sync_copy(ids_hbm.at[pl.ds(base, PER_W)], idx_v)

        def start_gather(s, b):
            pltpu.async_copy(
                table_hbm.at[idx_v.at[pl.ds(s * SUP, SUP)]], rbs[b], gss[b]
            )

        def wait_gather(b):
            pltpu.make_async_copy(
                table_hbm.at[idx_v.at[pl.ds(0, SUP)]], rbs[b], gss[b]
            ).wait()

        def wait_out(vb):
            pltpu.make_async_copy(
                vts[vb].at[:, pl.ds(0, 128)], out_hbm.at[0, :, 0], oss[vb]
            ).wait()

        def transpose_block(rb, blk, vtb):
            for r in range(128):
                row = blk * 128 + r
                cidx = zero16 + r
                lo = rb[row, pl.ds(0, 16)]
                hi = rb[row, pl.ds(16, 16)]
                plsc.store_scatter(vtb, [iota, cidx], lo)
                plsc.store_scatter(vtb, [iota16p, cidx], hi)

        start_gather(0, 0)

        def outer(s2, carry):
            for b in range(2):
                s = s2 * 2 + b

                @pl.when(s + 1 < N_SUP)
                def _():
                    start_gather(s + 1, 1 - b)

                wait_gather(b)
                for blk in range(BLKS):
                    vb = blk & 1
                    if blk < 2:
                        @pl.when(s > 0)
                        def _():
                            wait_out(vb)
                    else:
                        wait_out(vb)
                    transpose_block(rbs[b], blk, vts[vb])
                    g = w * (PER_W // 128) + s * BLKS + blk
                    j = g >> 5
                    ih = g & 31
                    for dh in range(4):
                        pltpu.async_copy(
                            vts[vb].at[pl.ds(dh * 8, 8), pl.ds(0, 128)],
                            out_hbm.at[j, dh, ih],
                            oss[vb],
                        )
            return carry

        lax.fori_loop(0, N_SUP // 2, outer, 0)
        wait_out(0)
        wait_out(1)

    return k(ids5, table)


def kernel(token_ids, weight):
    ids5 = jnp.transpose(token_ids).reshape(B)
    out5 = _gather(ids5, weight)
    return out5.transpose(2, 4, 0, 1, 3).reshape(NI, NJ, D)
